# Initial kernel scaffold; baseline (speedup 1.0000x reference)
import jax
import jax.numpy as jnp
from jax.experimental import pallas as pl


def kernel(x, pos, gnn_W, gnn_b, mlp_W1, mlp_b1, mlp_W2, mlp_b2, edge_index, batch):
    raise NotImplementedError("write your pallas kernel here")


# speedup vs baseline: 2.2479x; 2.2479x over previous
import functools

import jax
import jax.numpy as jnp
from jax import lax
from jax.experimental import pallas as pl
from jax.experimental.pallas import tpu as pltpu
from jax.experimental.pallas import tpu_sc as plsc

N = 2048
E = 32768
D = 128
L = 3
G = 16
SIGMA_H = 2.0
SIGMA_L = 1.0
BFS_ITERS = 6

NW = 32
ROWS = 32
EB = 8192
RB = 256


def _sc_adj_body(dst_hbm, src_hbm, c_hbm, ct_hbm, acc, keybuf, othbuf):
    wid = lax.axis_index("s") * 2 + lax.axis_index("c")
    zeros16 = jnp.zeros((16,), jnp.float32)
    ones16 = jnp.ones((16,), jnp.float32)

    for phase in range(2):
        out_hbm = c_hbm if phase == 0 else ct_hbm
        key_hbm = dst_hbm if phase == 0 else src_hbm
        oth_hbm = src_hbm if phase == 0 else dst_hbm
        for half in range(2):
            r0 = wid * (2 * ROWS) + half * ROWS

            def zbody(j, _):
                base = j * 128
                for u in range(8):
                    acc[pl.ds(base + u * 16, 16)] = zeros16
                return 0

            lax.fori_loop(0, (ROWS * N) // 128, zbody, 0)

            for ch in range(E // EB):
                pltpu.sync_copy(key_hbm.at[pl.ds(ch * EB, EB)], keybuf)
                pltpu.sync_copy(oth_hbm.at[pl.ds(ch * EB, EB)], othbuf)

                def ebody(j, _):
                    kv = keybuf[pl.ds(j * 16, 16)]
                    ov = othbuf[pl.ds(j * 16, 16)]
                    rel = kv - r0
                    m = (rel >= 0) & (rel < ROWS)
                    flat = rel * N + ov
                    flat = jnp.where(m, flat, 0)
                    plsc.addupdate_scatter(acc, [flat], ones16, mask=m)
                    return 0

                lax.fori_loop(0, EB // 16, ebody, 0)

            pltpu.sync_copy(acc, out_hbm.at[pl.ds(r0 * N, ROWS * N)])


def _sc_build_adj(dst, src):
    mesh = plsc.VectorSubcoreMesh(core_axis_name="c", subcore_axis_name="s")
    f = functools.partial(
        pl.kernel,
        out_type=(
            jax.ShapeDtypeStruct((N * N,), jnp.float32),
            jax.ShapeDtypeStruct((N * N,), jnp.float32),
        ),
        mesh=mesh,
        scratch_types=[
            pltpu.VMEM((ROWS * N,), jnp.float32),
            pltpu.VMEM((EB,), jnp.int32),
            pltpu.VMEM((EB,), jnp.int32),
        ],
        compiler_params=pltpu.CompilerParams(needs_layout_passes=False),
    )(_sc_adj_body)
    c_flat, ct_flat = f(dst, src)
    return c_flat.reshape(N, N), ct_flat.reshape(N, N)


def _prep_body(c_ref, ct_ref, mb_ref, dist_ref):
    i = pl.program_id(0)
    s = c_ref[...] + ct_ref[...]
    row = i * RB + lax.broadcasted_iota(jnp.int32, (RB, N), 0)
    col = lax.broadcasted_iota(jnp.int32, (RB, N), 1)
    eye = row == col
    adj = s > 0.0
    mb_ref[...] = jnp.where(adj | eye, 1.0, 0.0).astype(jnp.bfloat16)
    dist_ref[...] = jnp.where(
        eye, 0.0, jnp.where(adj, 1.0, jnp.inf)
    ).astype(jnp.bfloat16)


def _prep(c, ct):
    return pl.pallas_call(
        _prep_body,
        grid=(N // RB,),
        in_specs=[
            pl.BlockSpec((RB, N), lambda i: (i, 0)),
            pl.BlockSpec((RB, N), lambda i: (i, 0)),
        ],
        out_specs=[
            pl.BlockSpec((RB, N), lambda i: (i, 0)),
            pl.BlockSpec((RB, N), lambda i: (i, 0)),
        ],
        out_shape=[
            jax.ShapeDtypeStruct((N, N), jnp.bfloat16),
            jax.ShapeDtypeStruct((N, N), jnp.bfloat16),
        ],
    )(c, ct)


def _gnn_body(c_ref, h_ref, w_ref, b_ref, out_ref, *, relu):
    i = pl.program_id(0)
    cb = c_ref[...]
    h_full = h_ref[...]
    agg = jnp.dot(cb, h_full, preferred_element_type=jnp.float32)
    deg = jnp.sum(cb, axis=1, keepdims=True) + 1.0
    hb = h_ref[pl.ds(i * RB, RB), :]
    z = (hb + agg) / deg
    out = jnp.dot(z, w_ref[...], preferred_element_type=jnp.float32) + b_ref[...]
    if relu:
        out = jnp.maximum(out, 0.0)
    out_ref[...] = out


def _gnn_layer(c, h, w, b, relu):
    return pl.pallas_call(
        functools.partial(_gnn_body, relu=relu),
        grid=(N // RB,),
        in_specs=[
            pl.BlockSpec((RB, N), lambda i: (i, 0)),
            pl.BlockSpec((N, D), lambda i: (0, 0)),
            pl.BlockSpec((D, D), lambda i: (0, 0)),
            pl.BlockSpec((1, D), lambda i: (0, 0)),
        ],
        out_specs=pl.BlockSpec((RB, D), lambda i: (i, 0)),
        out_shape=jax.ShapeDtypeStruct((N, D), jnp.float32),
    )(c, h, w, b)


def _bfs_body(mb_ref, dist_ref, dist_out, *, kval):
    dold = dist_ref[...].astype(jnp.float32)
    reach = jnp.where(dold <= (kval - 1.0), 1.0, 0.0).astype(jnp.bfloat16)
    prod = lax.dot_general(
        reach, mb_ref[...], (((1,), (0,)), ((), ())),
        preferred_element_type=jnp.float32,
    )
    new = prod > 0.0
    dist_out[...] = jnp.where(
        new & (dold == jnp.inf), jnp.float32(kval), dold
    ).astype(jnp.bfloat16)


def _bfs(mb, dist0):
    dist = dist0
    for k in range(2, BFS_ITERS + 1):
        dist = pl.pallas_call(
            functools.partial(_bfs_body, kval=float(k)),
            grid=(N // RB,),
            in_specs=[
                pl.BlockSpec((N, N), lambda i: (0, 0)),
                pl.BlockSpec((RB, N), lambda i: (i, 0)),
            ],
            out_specs=pl.BlockSpec((RB, N), lambda i: (i, 0)),
            out_shape=jax.ShapeDtypeStruct((N, N), jnp.bfloat16),
        )(mb, dist)
    return dist


def _head_body(nf_ref, batch_ref, pos_ref, w1_ref, b1_ref, w2_ref, b2_ref,
               pred_ref, gf_ref, ploss_ref):
    nf = nf_ref[...]
    hidden = jnp.maximum(
        jnp.dot(nf, w1_ref[...], preferred_element_type=jnp.float32)
        + b1_ref[...], 0.0)
    pred = jnp.dot(hidden, w2_ref[...], preferred_element_type=jnp.float32) \
        + b2_ref[...]
    pred_ref[...] = pred
    seg = lax.broadcasted_iota(jnp.int32, (G, N), 0)
    onehot = jnp.where(batch_ref[...] == seg, 1.0, 0.0)
    counts = jnp.sum(onehot, axis=1, keepdims=True)
    gf_ref[...] = jnp.dot(onehot, nf, preferred_element_type=jnp.float32) \
        / jnp.maximum(counts, 1.0)
    d = pred - pos_ref[...]
    ploss_ref[...] = (jnp.sum(d * d) / (N * 3)).reshape(1, 1)


def _head(nf, batch2d, pos, w1, b1, w2, b2):
    return pl.pallas_call(
        _head_body,
        grid=(1,),
        in_specs=[
            pl.BlockSpec((N, D), lambda i: (0, 0)),
            pl.BlockSpec((1, N), lambda i: (0, 0)),
            pl.BlockSpec((N, 3), lambda i: (0, 0)),
            pl.BlockSpec((D, D), lambda i: (0, 0)),
            pl.BlockSpec((1, D), lambda i: (0, 0)),
            pl.BlockSpec((D, 3), lambda i: (0, 0)),
            pl.BlockSpec((1, 3), lambda i: (0, 0)),
        ],
        out_specs=[
            pl.BlockSpec((N, 3), lambda i: (0, 0)),
            pl.BlockSpec((G, D), lambda i: (0, 0)),
            pl.BlockSpec((1, 1), lambda i: (0, 0)),
        ],
        out_shape=[
            jax.ShapeDtypeStruct((N, 3), jnp.float32),
            jax.ShapeDtypeStruct((G, D), jnp.float32),
            jax.ShapeDtypeStruct((1, 1), jnp.float32),
        ],
    )(nf, batch2d, pos, w1, b1, w2, b2)


def _mani_body(dist_ref, pp_ref, ppt_ref, out_ref):
    i = pl.program_id(0)
    acc = jnp.zeros((RB, N), jnp.float32)
    for c in range(3):
        colv = pp_ref[:, c:c + 1]
        rowv = ppt_ref[c:c + 1, :]
        dc = colv - rowv
        acc = acc + dc * dc
    dl = jnp.sqrt(acc + 1e-12)
    wl = jnp.exp(-dl / SIGMA_L)
    wh = jnp.exp(-dist_ref[...].astype(jnp.float32) / SIGMA_H)
    r = wh - wl
    part = jnp.sum(r * r).reshape(1, 1)
    out_ref[...] = jnp.where(i == 0, part, out_ref[...] + part)


def _mani(dist, pred_pos, pred_pos_t):
    return pl.pallas_call(
        _mani_body,
        grid=(N // RB,),
        in_specs=[
            pl.BlockSpec((RB, N), lambda i: (i, 0)),
            pl.BlockSpec((RB, 3), lambda i: (i, 0)),
            pl.BlockSpec((3, N), lambda i: (0, 0)),
        ],
        out_specs=pl.BlockSpec((1, 1), lambda i: (0, 0)),
        out_shape=jax.ShapeDtypeStruct((1, 1), jnp.float32),
        compiler_params=pltpu.CompilerParams(
            dimension_semantics=("arbitrary",)
        ),
    )(dist, pred_pos, pred_pos_t)


def kernel(x, pos, gnn_W, gnn_b, mlp_W1, mlp_b1, mlp_W2, mlp_b2,
           edge_index, batch):
    src = edge_index[0]
    dst = edge_index[1]

    c, ct = _sc_build_adj(dst, src)
    mb, dist0 = _prep(c, ct)

    h = x
    for l in range(L):
        h = _gnn_layer(c, h, gnn_W[l], gnn_b[l].reshape(1, D), relu=(l < L - 1))

    dist = _bfs(mb, dist0)

    pred_pos, graph_feat, ploss = _head(
        h, batch.reshape(1, N), pos, mlp_W1, mlp_b1.reshape(1, D),
        mlp_W2, mlp_b2.reshape(1, 3))

    mani = _mani(dist, pred_pos, pred_pos.T)

    return (pred_pos, graph_feat, ploss.reshape(()), mani.reshape(()))


# speedup vs baseline: 2.4462x; 1.0882x over previous
import functools

import jax
import jax.numpy as jnp
from jax import lax
from jax.experimental import pallas as pl
from jax.experimental.pallas import tpu as pltpu
from jax.experimental.pallas import tpu_sc as plsc

N = 2048
E = 32768
D = 128
L = 3
G = 16
SIGMA_H = 2.0
SIGMA_L = 1.0
BFS_ITERS = 6

NW = 32
ROWS = 32
EB = 8192
RB = 256


def _sc_counts_body(key_hbm, oth_hbm, out_hbm, acc, keybuf, othbuf,
                    ks0, ks1, os0, os1):
    wid = lax.axis_index("s") * 2 + lax.axis_index("c")
    zeros16 = jnp.zeros((16,), jnp.float32)
    ones16 = jnp.ones((16,), jnp.float32)
    ksems = (ks0, ks1)
    osems = (os0, os1)
    nch = E // EB

    def _start(ch):
        b = ch % 2
        kcp = pltpu.make_async_copy(
            key_hbm.at[pl.ds(ch * EB, EB)], keybuf.at[b], ksems[b])
        ocp = pltpu.make_async_copy(
            oth_hbm.at[pl.ds(ch * EB, EB)], othbuf.at[b], osems[b])
        kcp.start()
        ocp.start()
        return kcp, ocp

    for half in range(2):
        r0 = wid * (2 * ROWS) + half * ROWS

        cps = {0: _start(0)}

        def zbody(j, _):
            base = j * 128
            for u in range(8):
                acc[pl.ds(base + u * 16, 16)] = zeros16
            return 0

        lax.fori_loop(0, (ROWS * N) // 128, zbody, 0)

        for ch in range(nch):
            b = ch % 2
            if ch + 1 < nch:
                cps[ch + 1] = _start(ch + 1)
            kcp, ocp = cps.pop(ch)
            kcp.wait()
            ocp.wait()

            def ebody(j, _):
                base = j * 32
                for u in range(2):
                    kv = keybuf[b, pl.ds(base + u * 16, 16)]
                    ov = othbuf[b, pl.ds(base + u * 16, 16)]
                    rel = kv - r0
                    m = (rel >= 0) & (rel < ROWS)
                    flat = jnp.where(m, rel * N + ov, 0)
                    plsc.addupdate_scatter(acc, [flat], ones16, mask=m)
                return 0

            lax.fori_loop(0, EB // 32, ebody, 0)

        pltpu.sync_copy(acc, out_hbm.at[pl.ds(r0 * N, ROWS * N)])


def _sc_counts(key, oth):
    mesh = plsc.VectorSubcoreMesh(core_axis_name="c", subcore_axis_name="s")
    f = functools.partial(
        pl.kernel,
        out_type=jax.ShapeDtypeStruct((N * N,), jnp.float32),
        mesh=mesh,
        scratch_types=[
            pltpu.VMEM((ROWS * N,), jnp.float32),
            pltpu.VMEM((2, EB), jnp.int32),
            pltpu.VMEM((2, EB), jnp.int32),
            pltpu.SemaphoreType.DMA,
            pltpu.SemaphoreType.DMA,
            pltpu.SemaphoreType.DMA,
            pltpu.SemaphoreType.DMA,
        ],
        compiler_params=pltpu.CompilerParams(needs_layout_passes=False),
    )(_sc_counts_body)
    return f(key, oth).reshape(N, N)


def _prep_body(c_ref, ct_ref, mb_ref, dist_ref):
    i = pl.program_id(0)
    s = c_ref[...] + ct_ref[...]
    row = i * RB + lax.broadcasted_iota(jnp.int32, (RB, N), 0)
    col = lax.broadcasted_iota(jnp.int32, (RB, N), 1)
    eye = row == col
    adj = s > 0.0
    mb_ref[...] = jnp.where(adj | eye, 1.0, 0.0).astype(jnp.bfloat16)
    dist_ref[...] = jnp.where(
        eye, 0.0, jnp.where(adj, 1.0, jnp.inf)
    ).astype(jnp.bfloat16)


def _prep(c, ct):
    return pl.pallas_call(
        _prep_body,
        grid=(N // RB,),
        in_specs=[
            pl.BlockSpec((RB, N), lambda i: (i, 0)),
            pl.BlockSpec((RB, N), lambda i: (i, 0)),
        ],
        out_specs=[
            pl.BlockSpec((RB, N), lambda i: (i, 0)),
            pl.BlockSpec((RB, N), lambda i: (i, 0)),
        ],
        out_shape=[
            jax.ShapeDtypeStruct((N, N), jnp.bfloat16),
            jax.ShapeDtypeStruct((N, N), jnp.bfloat16),
        ],
    )(c, ct)


def _gnn_body(c_ref, h_ref, w_ref, b_ref, out_ref, *, relu):
    i = pl.program_id(0)
    cb = c_ref[...]
    h_full = h_ref[...]
    agg = jnp.dot(cb, h_full, preferred_element_type=jnp.float32)
    deg = jnp.sum(cb, axis=1, keepdims=True) + 1.0
    hb = h_ref[pl.ds(i * RB, RB), :]
    z = (hb + agg) / deg
    out = jnp.dot(z, w_ref[...], preferred_element_type=jnp.float32) + b_ref[...]
    if relu:
        out = jnp.maximum(out, 0.0)
    out_ref[...] = out


def _gnn_layer(c, h, w, b, relu):
    return pl.pallas_call(
        functools.partial(_gnn_body, relu=relu),
        grid=(N // RB,),
        in_specs=[
            pl.BlockSpec((RB, N), lambda i: (i, 0)),
            pl.BlockSpec((N, D), lambda i: (0, 0)),
            pl.BlockSpec((D, D), lambda i: (0, 0)),
            pl.BlockSpec((1, D), lambda i: (0, 0)),
        ],
        out_specs=pl.BlockSpec((RB, D), lambda i: (i, 0)),
        out_shape=jax.ShapeDtypeStruct((N, D), jnp.float32),
    )(c, h, w, b)


def _bfs_body(mb_ref, dist_ref, dist_out, *, kval):
    dold = dist_ref[...].astype(jnp.float32)
    reach = jnp.where(dold <= (kval - 1.0), 1.0, 0.0).astype(jnp.bfloat16)
    prod = lax.dot_general(
        reach, mb_ref[...], (((1,), (0,)), ((), ())),
        preferred_element_type=jnp.float32,
    )
    new = prod > 0.0
    dist_out[...] = jnp.where(
        new & (dold == jnp.inf), jnp.float32(kval), dold
    ).astype(jnp.bfloat16)


def _bfs(mb, dist0):
    dist = dist0
    for k in range(2, BFS_ITERS + 1):
        dist = pl.pallas_call(
            functools.partial(_bfs_body, kval=float(k)),
            grid=(N // RB,),
            in_specs=[
                pl.BlockSpec((N, N), lambda i: (0, 0)),
                pl.BlockSpec((RB, N), lambda i: (i, 0)),
            ],
            out_specs=pl.BlockSpec((RB, N), lambda i: (i, 0)),
            out_shape=jax.ShapeDtypeStruct((N, N), jnp.bfloat16),
        )(mb, dist)
    return dist


def _head_body(nf_ref, batch_ref, pos_ref, w1_ref, b1_ref, w2_ref, b2_ref,
               pred_ref, gf_ref, ploss_ref):
    nf = nf_ref[...]
    hidden = jnp.maximum(
        jnp.dot(nf, w1_ref[...], preferred_element_type=jnp.float32)
        + b1_ref[...], 0.0)
    pred = jnp.dot(hidden, w2_ref[...], preferred_element_type=jnp.float32) \
        + b2_ref[...]
    pred_ref[...] = pred
    seg = lax.broadcasted_iota(jnp.int32, (G, N), 0)
    onehot = jnp.where(batch_ref[...] == seg, 1.0, 0.0)
    counts = jnp.sum(onehot, axis=1, keepdims=True)
    gf_ref[...] = jnp.dot(onehot, nf, preferred_element_type=jnp.float32) \
        / jnp.maximum(counts, 1.0)
    d = pred - pos_ref[...]
    ploss_ref[...] = (jnp.sum(d * d) / (N * 3)).reshape(1, 1)


def _head(nf, batch2d, pos, w1, b1, w2, b2):
    return pl.pallas_call(
        _head_body,
        grid=(1,),
        in_specs=[
            pl.BlockSpec((N, D), lambda i: (0, 0)),
            pl.BlockSpec((1, N), lambda i: (0, 0)),
            pl.BlockSpec((N, 3), lambda i: (0, 0)),
            pl.BlockSpec((D, D), lambda i: (0, 0)),
            pl.BlockSpec((1, D), lambda i: (0, 0)),
            pl.BlockSpec((D, 3), lambda i: (0, 0)),
            pl.BlockSpec((1, 3), lambda i: (0, 0)),
        ],
        out_specs=[
            pl.BlockSpec((N, 3), lambda i: (0, 0)),
            pl.BlockSpec((G, D), lambda i: (0, 0)),
            pl.BlockSpec((1, 1), lambda i: (0, 0)),
        ],
        out_shape=[
            jax.ShapeDtypeStruct((N, 3), jnp.float32),
            jax.ShapeDtypeStruct((G, D), jnp.float32),
            jax.ShapeDtypeStruct((1, 1), jnp.float32),
        ],
    )(nf, batch2d, pos, w1, b1, w2, b2)


def _mani_body(dist_ref, pp_ref, ppt_ref, out_ref):
    i = pl.program_id(0)
    acc = jnp.zeros((RB, N), jnp.float32)
    for c in range(3):
        colv = pp_ref[:, c:c + 1]
        rowv = ppt_ref[c:c + 1, :]
        dc = colv - rowv
        acc = acc + dc * dc
    dl = jnp.sqrt(acc + 1e-12)
    wl = jnp.exp(-dl / SIGMA_L)
    wh = jnp.exp(-dist_ref[...].astype(jnp.float32) / SIGMA_H)
    r = wh - wl
    part = jnp.sum(r * r).reshape(1, 1)
    out_ref[...] = jnp.where(i == 0, part, out_ref[...] + part)


def _mani(dist, pred_pos, pred_pos_t):
    return pl.pallas_call(
        _mani_body,
        grid=(N // RB,),
        in_specs=[
            pl.BlockSpec((RB, N), lambda i: (i, 0)),
            pl.BlockSpec((RB, 3), lambda i: (i, 0)),
            pl.BlockSpec((3, N), lambda i: (0, 0)),
        ],
        out_specs=pl.BlockSpec((1, 1), lambda i: (0, 0)),
        out_shape=jax.ShapeDtypeStruct((1, 1), jnp.float32),
        compiler_params=pltpu.CompilerParams(
            dimension_semantics=("arbitrary",)
        ),
    )(dist, pred_pos, pred_pos_t)


def kernel(x, pos, gnn_W, gnn_b, mlp_W1, mlp_b1, mlp_W2, mlp_b2,
           edge_index, batch):
    src = edge_index[0]
    dst = edge_index[1]

    c = _sc_counts(dst, src)
    ct = _sc_counts(src, dst)
    mb, dist0 = _prep(c, ct)

    h = x
    for l in range(L):
        h = _gnn_layer(c, h, gnn_W[l], gnn_b[l].reshape(1, D), relu=(l < L - 1))

    dist = _bfs(mb, dist0)

    pred_pos, graph_feat, ploss = _head(
        h, batch.reshape(1, N), pos, mlp_W1, mlp_b1.reshape(1, D),
        mlp_W2, mlp_b2.reshape(1, 3))

    mani = _mani(dist, pred_pos, pred_pos.T)

    return (pred_pos, graph_feat, ploss.reshape(()), mani.reshape(()))
